# baseline (speedup 1.0000x reference)
import jax
import jax.numpy as jnp
from jax.experimental import pallas as pl
from jax.experimental.pallas import tpu as pltpu


def _rvq_kernel(z_ref, books_ref, out_ref):
    r = z_ref[0]
    acc = jnp.zeros_like(r)
    n_books, C, D = books_ref.shape
    iota_c = jax.lax.broadcasted_iota(jnp.int32, (C, 1), 0)
    for i in range(n_books):
        emb = books_ref[i]
        scores = jax.lax.dot_general(
            emb.astype(jnp.bfloat16), r.astype(jnp.bfloat16),
            (((1,), (0,)), ((), ())),
            preferred_element_type=jnp.float32)
        scores = scores - 0.5 * jnp.sum(emb * emb, axis=1, keepdims=True)
        idx = jnp.argmax(scores, axis=0)
        oh = (iota_c == idx[None, :]).astype(jnp.float32)
        q = jax.lax.dot_general(
            emb, oh, (((0,), (0,)), ((), ())),
            precision=jax.lax.Precision.HIGHEST,
            preferred_element_type=jnp.float32)
        acc = acc + q
        r = r - q
    out_ref[0] = acc


def kernel(z, books):
    B, D, T = z.shape
    n_books, C, Dk = books.shape
    grid = (B,)
    return pl.pallas_call(
        _rvq_kernel,
        grid=grid,
        in_specs=[
            pl.BlockSpec((1, D, T), lambda b: (b, 0, 0)),
            pl.BlockSpec((n_books, C, Dk), lambda b: (0, 0, 0)),
        ],
        out_specs=pl.BlockSpec((1, D, T), lambda b: (b, 0, 0)),
        out_shape=jax.ShapeDtypeStruct((B, D, T), z.dtype),
        compiler_params=pltpu.CompilerParams(
            dimension_semantics=("parallel",),
        ),
    )(z, books)


# speedup vs baseline: 1.7146x; 1.7146x over previous
import jax
import jax.numpy as jnp
from jax.experimental import pallas as pl
from jax.experimental.pallas import tpu as pltpu


def _rvq_kernel(z_ref, books_ref, out_ref):
    x = z_ref[0]
    r = x
    n_books, C, D = books_ref.shape
    iota_c = jax.lax.broadcasted_iota(jnp.int32, (C, 1), 0)
    bf16, f32 = jnp.bfloat16, jnp.float32
    for i in range(n_books):
        emb = books_ref[i]
        scores = jax.lax.dot_general(
            emb.astype(bf16), r.astype(bf16),
            (((1,), (0,)), ((), ())),
            preferred_element_type=f32)
        scores = scores - 0.5 * jnp.sum(emb * emb, axis=1, keepdims=True)
        idx = jnp.argmax(scores, axis=0)
        oh = (iota_c == idx[None, :]).astype(bf16)
        e_hi = emb.astype(bf16)
        r1 = emb - e_hi.astype(f32)
        e_mid = r1.astype(bf16)
        e_lo = (r1 - e_mid.astype(f32)).astype(bf16)
        q = jax.lax.dot_general(e_hi, oh, (((0,), (0,)), ((), ())),
                                preferred_element_type=f32)
        q = q + jax.lax.dot_general(e_mid, oh, (((0,), (0,)), ((), ())),
                                    preferred_element_type=f32)
        q = q + jax.lax.dot_general(e_lo, oh, (((0,), (0,)), ((), ())),
                                    preferred_element_type=f32)
        r = r - q
    out_ref[0] = x - r


def kernel(z, books):
    B, D, T = z.shape
    n_books, C, Dk = books.shape
    grid = (B,)
    return pl.pallas_call(
        _rvq_kernel,
        grid=grid,
        in_specs=[
            pl.BlockSpec((1, D, T), lambda b: (b, 0, 0)),
            pl.BlockSpec((n_books, C, Dk), lambda b: (0, 0, 0)),
        ],
        out_specs=pl.BlockSpec((1, D, T), lambda b: (b, 0, 0)),
        out_shape=jax.ShapeDtypeStruct((B, D, T), z.dtype),
        compiler_params=pltpu.CompilerParams(
            dimension_semantics=("parallel",),
        ),
    )(z, books)
